# baseline (speedup 1.0000x reference)
import jax
import jax.numpy as jnp
from jax.experimental import pallas as pl

MAX_LEN = 8192


def _add_kernel(x_ref, t_ref, o_ref):
    o_ref[...] = x_ref[...] + t_ref[...]


def kernel(x, table):
    bsz, seq_len, d = x.shape
    if seq_len > MAX_LEN:
        x = x[:, -MAX_LEN:, :]
        seq_len = MAX_LEN
    bs = 512
    grid = (seq_len // bs,)
    return pl.pallas_call(
        _add_kernel,
        grid=grid,
        in_specs=[
            pl.BlockSpec((bsz, bs, d), lambda j: (0, j, 0)),
            pl.BlockSpec((bs, d), lambda j: (j, 0)),
        ],
        out_specs=pl.BlockSpec((bsz, bs, d), lambda j: (0, j, 0)),
        out_shape=jax.ShapeDtypeStruct(x.shape, x.dtype),
    )(x, table)
